# baseline (speedup 1.0000x reference)
import functools

import jax
import jax.numpy as jnp
from jax import lax
from jax.experimental import pallas as pl
from jax.experimental.pallas import tpu as pltpu
from jax.experimental.pallas import tpu_sc as plsc

NUM_FRAMES = 1024
NUM_TYPES = 4
NUM_ACTIONS = 128
TOTAL_CLS = 256
ROW = NUM_ACTIONS * TOTAL_CLS
NW = 32
FPW = NUM_FRAMES // NW
AW = NUM_TYPES * NUM_ACTIONS
NBUF = 2


def _sc_body(a_hbm, z_hbm, out_hbm, a_v, b0, b1, s0, s1, sz):
    wid = lax.axis_index("s") * 2 + lax.axis_index("c")
    fbase = wid * FPW
    bufs = (b0, b1)
    sems = (s0, s1)
    ca = pltpu.async_copy(a_hbm.at[pl.ds(fbase * AW, FPW * AW)], a_v, sz)
    cz = [
        pltpu.async_copy(z_hbm, bufs[b], sems[b]) for b in range(NBUF)
    ]
    ca.wait()
    for c in cz:
        c.wait()

    siota = lax.iota(jnp.int32, 16) * TOTAL_CLS
    ones = jnp.ones((16,), jnp.float32)
    zeros = jnp.zeros((16,), jnp.float32)

    def put(buf, i, val):
        for t in range(NUM_TYPES):
            for ch in range(NUM_ACTIONS // 16):
                av = a_v[pl.ds(i * AW + t * NUM_ACTIONS + ch * 16, 16)]
                idx = av + (siota + (ch * 16 * TOTAL_CLS + t * 64))
                plsc.store_scatter(buf, [idx], val)

    def advance(b, i):
        pltpu.make_async_copy(
            bufs[b], out_hbm.at[fbase + i - NBUF], sems[b]
        ).wait()
        put(bufs[b], i - NBUF, zeros)
        put(bufs[b], i, ones)
        pltpu.async_copy(bufs[b], out_hbm.at[fbase + i], sems[b])

    for b in range(NBUF):
        put(bufs[b], b, ones)
        pltpu.async_copy(bufs[b], out_hbm.at[fbase + b], sems[b])

    def step(k, _):
        g = NBUF * k
        for b in range(NBUF):
            advance(b, g + b)
        return _

    nfull = (FPW - NBUF) // NBUF
    lax.fori_loop(1, 1 + nfull, step, 0)

    done = NBUF + nfull * NBUF
    for i in range(done, FPW):
        advance(i % NBUF, i)
    for i in range(FPW - NBUF, FPW):
        pltpu.make_async_copy(
            bufs[i % NBUF], out_hbm.at[fbase + i], sems[i % NBUF]
        ).wait()


def kernel(actions):
    mesh = plsc.VectorSubcoreMesh(core_axis_name="c", subcore_axis_name="s")
    sck = functools.partial(
        pl.kernel,
        out_type=jax.ShapeDtypeStruct((NUM_FRAMES, ROW), jnp.float32),
        mesh=mesh,
        scratch_types=[
            pltpu.VMEM((FPW * AW,), jnp.int32),
            pltpu.VMEM((ROW,), jnp.float32),
            pltpu.VMEM((ROW,), jnp.float32),
            pltpu.SemaphoreType.DMA,
            pltpu.SemaphoreType.DMA,
            pltpu.SemaphoreType.DMA,
        ],
        compiler_params=pltpu.CompilerParams(needs_layout_passes=False),
    )(_sc_body)
    af = actions.reshape(NUM_FRAMES * AW)
    zrow = jnp.zeros((ROW,), jnp.float32)
    out = sck(af, zrow)
    return out.reshape(NUM_FRAMES, NUM_ACTIONS, TOTAL_CLS)
